# Initial kernel scaffold; baseline (speedup 1.0000x reference)
import jax
import jax.numpy as jnp
from jax.experimental import pallas as pl


def kernel(x, edge_index, batch, W_l0, W_r0, b0, W_l1, W_r1, b1, W_l2, W_r2, b2, W_fc1, b_fc1, W_fc2, b_fc2):
    raise NotImplementedError("write your pallas kernel here")


# speedup vs baseline: 2.2986x; 2.2986x over previous
import functools

import jax
import jax.numpy as jnp
from jax import lax
from jax.experimental import pallas as pl
from jax.experimental.pallas import tpu as pltpu
from jax.experimental.pallas import tpu_sc as plsc

LANES = 16


def _make_agg(P, N, E, n_workers=32, c_per=4, eb=16000):
    assert P % (n_workers * c_per) == 0 and E % eb == 0 and eb % LANES == 0
    npass = P // (n_workers * c_per)
    nblk = E // eb

    mesh = plsc.VectorSubcoreMesh(core_axis_name="c", subcore_axis_name="s")

    @functools.partial(
        pl.kernel,
        out_type=jax.ShapeDtypeStruct((P * N,), jnp.float32),
        mesh=mesh,
        compiler_params=pltpu.CompilerParams(needs_layout_passes=False),
        scratch_types=[
            pltpu.VMEM((c_per * N,), jnp.float32),
            pltpu.VMEM((c_per * N,), jnp.float32),
            pltpu.VMEM((eb,), jnp.int32),
            pltpu.VMEM((eb,), jnp.int32),
        ],
    )
    def agg(hT, src_hbm, dst_hbm, out, slab, acc, sbuf, dbuf):
        wid = lax.axis_index("s") * 2 + lax.axis_index("c")
        for p in range(npass):
            r0 = p * (n_workers * c_per) + wid * c_per
            pltpu.sync_copy(hT.at[pl.ds(r0 * N, c_per * N)], slab)

            def zbody(i, carry):
                acc[pl.ds(i * LANES, LANES)] = jnp.zeros((LANES,), jnp.float32)
                return carry
            lax.fori_loop(0, c_per * N // LANES, zbody, 0)

            def blk_body(b, carry):
                pltpu.sync_copy(src_hbm.at[pl.ds(b * eb, eb)], sbuf)
                pltpu.sync_copy(dst_hbm.at[pl.ds(b * eb, eb)], dbuf)

                def ed(i, c2):
                    sv = sbuf[pl.ds(i * LANES, LANES)]
                    dv = dbuf[pl.ds(i * LANES, LANES)]
                    for j in range(c_per):
                        g = plsc.load_gather(slab, [sv + (j * N)])
                        plsc.addupdate_scatter(acc, [dv + (j * N)], g)
                    return c2
                return lax.fori_loop(0, eb // LANES, ed, carry)
            lax.fori_loop(0, nblk, blk_body, 0)

            pltpu.sync_copy(acc, out.at[pl.ds(r0 * N, c_per * N)])

    return agg


def _make_deg(N, E, n_workers=32):
    assert E % (n_workers * LANES) == 0
    epw = E // n_workers

    mesh = plsc.VectorSubcoreMesh(core_axis_name="c", subcore_axis_name="s")

    @functools.partial(
        pl.kernel,
        out_type=jax.ShapeDtypeStruct((n_workers, N), jnp.float32),
        mesh=mesh,
        compiler_params=pltpu.CompilerParams(needs_layout_passes=False),
        scratch_types=[
            pltpu.VMEM((N,), jnp.float32),
            pltpu.VMEM((epw,), jnp.int32),
        ],
    )
    def deg(dst_hbm, out, dl, dbuf):
        wid = lax.axis_index("s") * 2 + lax.axis_index("c")

        def zbody(i, carry):
            dl[pl.ds(i * LANES, LANES)] = jnp.zeros((LANES,), jnp.float32)
            return carry
        lax.fori_loop(0, N // LANES, zbody, 0)

        pltpu.sync_copy(dst_hbm.at[pl.ds(wid * epw, epw)], dbuf)

        ones = jnp.full((LANES,), 1.0, jnp.float32)

        def ed(i, carry):
            dv = dbuf[pl.ds(i * LANES, LANES)]
            plsc.addupdate_scatter(dl, [dv], ones)
            return carry
        lax.fori_loop(0, epw // LANES, ed, 0)

        pltpu.sync_copy(dl, out.at[wid])

    return deg


def _contract00(a, b):
    return lax.dot_general(a, b, (((0,), (0,)), ((), ())),
                           preferred_element_type=jnp.float32)


def _transpose_tc(x):
    N, D = x.shape

    def body(x_ref, out_ref):
        i0 = lax.broadcasted_iota(jnp.int32, (D, D), 0)
        i1 = lax.broadcasted_iota(jnp.int32, (D, D), 1)
        eye = (i0 == i1).astype(jnp.float32)
        out_ref[...] = lax.dot_general(
            eye, x_ref[...], (((1,), (1,)), ((), ())),
            preferred_element_type=jnp.float32)

    return pl.pallas_call(
        body, out_shape=jax.ShapeDtypeStruct((D, N), jnp.float32))(x)


def _layer0_tc(aggT, xT, degs, W_l, W_r, b):
    Dd, N = xT.shape
    Hh = W_l.shape[1]

    def body(agg_ref, x_ref, degs_ref, wl_ref, wr_ref, b_ref, out_ref, inv_ref):
        deg = jnp.sum(degs_ref[...], axis=0, keepdims=True)
        inv = 1.0 / jnp.maximum(deg, 1.0)
        inv_ref[...] = inv
        aggn = agg_ref[...] * inv
        z = (_contract00(wl_ref[...], aggn)
             + _contract00(wr_ref[...], x_ref[...]) + b_ref[...])
        out_ref[...] = jnp.maximum(z, 0.0)

    return pl.pallas_call(
        body,
        out_shape=(jax.ShapeDtypeStruct((Hh, N), jnp.float32),
                   jax.ShapeDtypeStruct((1, N), jnp.float32)),
    )(aggT, xT, degs, W_l, W_r, b)


def _layer_tc(aggT, hT, inv_deg, W_l, W_r, b):
    Hh = W_l.shape[1]
    N = hT.shape[1]

    def body(agg_ref, h_ref, inv_ref, wl_ref, wr_ref, b_ref, out_ref):
        aggn = agg_ref[...] * inv_ref[...]
        z = (_contract00(wl_ref[...], aggn)
             + _contract00(wr_ref[...], h_ref[...]) + b_ref[...])
        out_ref[...] = jnp.maximum(z, 0.0)

    return pl.pallas_call(
        body, out_shape=jax.ShapeDtypeStruct((Hh, N), jnp.float32),
    )(aggT, hT, inv_deg, W_l, W_r, b)


def _pool_mlp_tc(h3T, batch2d, G, W_fc1, b_fc1, W_fc2, b_fc2):
    Hh, N = h3T.shape

    def body(h_ref, batch_ref, wf1_ref, bf1_ref, wf2_ref, bf2_ref, out_ref):
        gids = lax.broadcasted_iota(jnp.int32, (G, N), 0)
        oh = (batch_ref[...] == gids).astype(jnp.float32)
        pooledT = lax.dot_general(h_ref[...], oh, (((1,), (1,)), ((), ())),
                                  preferred_element_type=jnp.float32)
        counts = lax.dot_general(jnp.ones((1, N), jnp.float32), oh,
                                 (((1,), (1,)), ((), ())),
                                 preferred_element_type=jnp.float32)
        pooled = pooledT / jnp.maximum(counts, 1.0)
        z1 = jnp.maximum(_contract00(wf1_ref[...], pooled) + bf1_ref[...], 0.0)
        z2 = _contract00(wf2_ref[...], z1) + bf2_ref[...]
        out_ref[...] = 1.0 / (1.0 + jnp.exp(-z2))

    return pl.pallas_call(
        body, out_shape=jax.ShapeDtypeStruct((1, G), jnp.float32),
    )(h3T, batch2d, W_fc1, b_fc1, W_fc2, b_fc2)


def kernel(x, edge_index, batch, W_l0, W_r0, b0, W_l1, W_r1, b1,
           W_l2, W_r2, b2, W_fc1, b_fc1, W_fc2, b_fc2):
    N, D = x.shape
    E = edge_index.shape[1]
    H = W_l0.shape[1]
    G = 64

    src = edge_index[0]
    dst = edge_index[1]
    batch2d = batch.reshape(1, N)
    b0c = b0.reshape(H, 1)
    b1c = b1.reshape(H, 1)
    b2c = b2.reshape(H, 1)
    bf1 = b_fc1.reshape(-1, 1)
    bf2 = b_fc2.reshape(1, 1)

    agg_d = _make_agg(D, N, E)
    agg_h = _make_agg(H, N, E)
    deg_k = _make_deg(N, E)

    xT = _transpose_tc(x)
    degs = deg_k(dst)

    a0 = agg_d(xT.reshape(-1), src, dst).reshape(D, N)
    h1T, inv_deg = _layer0_tc(a0, xT, degs, W_l0, W_r0, b0c)

    a1 = agg_h(h1T.reshape(-1), src, dst).reshape(H, N)
    h2T = _layer_tc(a1, h1T, inv_deg, W_l1, W_r1, b1c)

    a2 = agg_h(h2T.reshape(-1), src, dst).reshape(H, N)
    h3T = _layer_tc(a2, h2T, inv_deg, W_l2, W_r2, b2c)

    out = _pool_mlp_tc(h3T, batch2d, G, W_fc1, bf1, W_fc2, bf2)
    return out.reshape(G)


# speedup vs baseline: 4.8768x; 2.1217x over previous
import functools

import jax
import jax.numpy as jnp
from jax import lax
from jax.experimental import pallas as pl
from jax.experimental.pallas import tpu as pltpu
from jax.experimental.pallas import tpu_sc as plsc

LANES = 16


def _make_agg(P, N, E, n_workers=32, c_per=4, eb=16000):
    assert P % (n_workers * c_per) == 0 and E % eb == 0 and eb % LANES == 0
    npass = P // (n_workers * c_per)
    nblk = E // eb

    mesh = plsc.VectorSubcoreMesh(core_axis_name="c", subcore_axis_name="s")

    @functools.partial(
        pl.kernel,
        out_type=jax.ShapeDtypeStruct((P * N,), jnp.float32),
        mesh=mesh,
        compiler_params=pltpu.CompilerParams(needs_layout_passes=False),
        scratch_types=[
            pltpu.VMEM((c_per * N,), jnp.float32),
            pltpu.VMEM((c_per * N,), jnp.float32),
            pltpu.VMEM((eb,), jnp.int32),
            pltpu.VMEM((eb,), jnp.int32),
        ],
    )
    def agg(hT, src_hbm, dst_hbm, out, slab, acc, sbuf, dbuf):
        wid = lax.axis_index("s") * 2 + lax.axis_index("c")
        for p in range(npass):
            r0 = p * (n_workers * c_per) + wid * c_per
            pltpu.sync_copy(hT.at[pl.ds(r0 * N, c_per * N)], slab)

            @plsc.parallel_loop(0, c_per * N // LANES, 1, unroll=8)
            def zbody(i):
                acc[pl.ds(i * LANES, LANES)] = jnp.zeros((LANES,), jnp.float32)

            def blk_body(b, carry):
                pltpu.sync_copy(src_hbm.at[pl.ds(b * eb, eb)], sbuf)
                pltpu.sync_copy(dst_hbm.at[pl.ds(b * eb, eb)], dbuf)

                @plsc.parallel_loop(0, eb // LANES, 1, unroll=8)
                def ed(i):
                    sv = sbuf[pl.ds(i * LANES, LANES)]
                    dv = dbuf[pl.ds(i * LANES, LANES)]
                    for j in range(c_per):
                        g = plsc.load_gather(slab, [sv + (j * N)])
                        plsc.addupdate_scatter(acc, [dv + (j * N)], g)
                return carry
            lax.fori_loop(0, nblk, blk_body, 0)

            pltpu.sync_copy(acc, out.at[pl.ds(r0 * N, c_per * N)])

    return agg


def _make_deg(N, E, n_workers=32):
    assert E % (n_workers * LANES) == 0
    epw = E // n_workers

    mesh = plsc.VectorSubcoreMesh(core_axis_name="c", subcore_axis_name="s")

    @functools.partial(
        pl.kernel,
        out_type=jax.ShapeDtypeStruct((n_workers, N), jnp.float32),
        mesh=mesh,
        compiler_params=pltpu.CompilerParams(needs_layout_passes=False),
        scratch_types=[
            pltpu.VMEM((N,), jnp.float32),
            pltpu.VMEM((epw,), jnp.int32),
        ],
    )
    def deg(dst_hbm, out, dl, dbuf):
        wid = lax.axis_index("s") * 2 + lax.axis_index("c")

        def zbody(i, carry):
            dl[pl.ds(i * LANES, LANES)] = jnp.zeros((LANES,), jnp.float32)
            return carry
        lax.fori_loop(0, N // LANES, zbody, 0)

        pltpu.sync_copy(dst_hbm.at[pl.ds(wid * epw, epw)], dbuf)

        ones = jnp.full((LANES,), 1.0, jnp.float32)

        def ed(i, carry):
            dv = dbuf[pl.ds(i * LANES, LANES)]
            plsc.addupdate_scatter(dl, [dv], ones)
            return carry
        lax.fori_loop(0, epw // LANES, ed, 0)

        pltpu.sync_copy(dl, out.at[wid])

    return deg


def _contract00(a, b):
    return lax.dot_general(a, b, (((0,), (0,)), ((), ())),
                           preferred_element_type=jnp.float32)


def _transpose_tc(x):
    N, D = x.shape

    def body(x_ref, out_ref):
        i0 = lax.broadcasted_iota(jnp.int32, (D, D), 0)
        i1 = lax.broadcasted_iota(jnp.int32, (D, D), 1)
        eye = (i0 == i1).astype(jnp.float32)
        out_ref[...] = lax.dot_general(
            eye, x_ref[...], (((1,), (1,)), ((), ())),
            preferred_element_type=jnp.float32)

    return pl.pallas_call(
        body, out_shape=jax.ShapeDtypeStruct((D, N), jnp.float32))(x)


def _layer0_tc(aggT, xT, degs, W_l, W_r, b):
    Dd, N = xT.shape
    Hh = W_l.shape[1]

    def body(agg_ref, x_ref, degs_ref, wl_ref, wr_ref, b_ref, out_ref, inv_ref):
        deg = jnp.sum(degs_ref[...], axis=0, keepdims=True)
        inv = 1.0 / jnp.maximum(deg, 1.0)
        inv_ref[...] = inv
        aggn = agg_ref[...] * inv
        z = (_contract00(wl_ref[...], aggn)
             + _contract00(wr_ref[...], x_ref[...]) + b_ref[...])
        out_ref[...] = jnp.maximum(z, 0.0)

    return pl.pallas_call(
        body,
        out_shape=(jax.ShapeDtypeStruct((Hh, N), jnp.float32),
                   jax.ShapeDtypeStruct((1, N), jnp.float32)),
    )(aggT, xT, degs, W_l, W_r, b)


def _layer_tc(aggT, hT, inv_deg, W_l, W_r, b):
    Hh = W_l.shape[1]
    N = hT.shape[1]

    def body(agg_ref, h_ref, inv_ref, wl_ref, wr_ref, b_ref, out_ref):
        aggn = agg_ref[...] * inv_ref[...]
        z = (_contract00(wl_ref[...], aggn)
             + _contract00(wr_ref[...], h_ref[...]) + b_ref[...])
        out_ref[...] = jnp.maximum(z, 0.0)

    return pl.pallas_call(
        body, out_shape=jax.ShapeDtypeStruct((Hh, N), jnp.float32),
    )(aggT, hT, inv_deg, W_l, W_r, b)


def _pool_mlp_tc(h3T, batch2d, G, W_fc1, b_fc1, W_fc2, b_fc2):
    Hh, N = h3T.shape

    def body(h_ref, batch_ref, wf1_ref, bf1_ref, wf2_ref, bf2_ref, out_ref):
        gids = lax.broadcasted_iota(jnp.int32, (G, N), 0)
        oh = (batch_ref[...] == gids).astype(jnp.float32)
        pooledT = lax.dot_general(h_ref[...], oh, (((1,), (1,)), ((), ())),
                                  preferred_element_type=jnp.float32)
        counts = lax.dot_general(jnp.ones((1, N), jnp.float32), oh,
                                 (((1,), (1,)), ((), ())),
                                 preferred_element_type=jnp.float32)
        pooled = pooledT / jnp.maximum(counts, 1.0)
        z1 = jnp.maximum(_contract00(wf1_ref[...], pooled) + bf1_ref[...], 0.0)
        z2 = _contract00(wf2_ref[...], z1) + bf2_ref[...]
        out_ref[...] = 1.0 / (1.0 + jnp.exp(-z2))

    return pl.pallas_call(
        body, out_shape=jax.ShapeDtypeStruct((1, G), jnp.float32),
    )(h3T, batch2d, W_fc1, b_fc1, W_fc2, b_fc2)


def kernel(x, edge_index, batch, W_l0, W_r0, b0, W_l1, W_r1, b1,
           W_l2, W_r2, b2, W_fc1, b_fc1, W_fc2, b_fc2):
    N, D = x.shape
    E = edge_index.shape[1]
    H = W_l0.shape[1]
    G = 64

    src = edge_index[0]
    dst = edge_index[1]
    batch2d = batch.reshape(1, N)
    b0c = b0.reshape(H, 1)
    b1c = b1.reshape(H, 1)
    b2c = b2.reshape(H, 1)
    bf1 = b_fc1.reshape(-1, 1)
    bf2 = b_fc2.reshape(1, 1)

    agg_d = _make_agg(D, N, E)
    agg_h = _make_agg(H, N, E)
    deg_k = _make_deg(N, E)

    xT = _transpose_tc(x)
    degs = deg_k(dst)

    a0 = agg_d(xT.reshape(-1), src, dst).reshape(D, N)
    h1T, inv_deg = _layer0_tc(a0, xT, degs, W_l0, W_r0, b0c)

    a1 = agg_h(h1T.reshape(-1), src, dst).reshape(H, N)
    h2T = _layer_tc(a1, h1T, inv_deg, W_l1, W_r1, b1c)

    a2 = agg_h(h2T.reshape(-1), src, dst).reshape(H, N)
    h3T = _layer_tc(a2, h2T, inv_deg, W_l2, W_r2, b2c)

    out = _pool_mlp_tc(h3T, batch2d, G, W_fc1, bf1, W_fc2, bf2)
    return out.reshape(G)
